# Initial kernel scaffold; baseline (speedup 1.0000x reference)
import jax
import jax.numpy as jnp
from jax.experimental import pallas as pl


def kernel(x, edge_index, edge_attr, W_mem, b_mem, g_mln, b_mln, Wq, Wk, Wv, Wo, W_eu, b_eu, g_eln, b_eln, g_en, b_en, W_f1, b_f1, W_f2, b_f2, g_n1, b_n1, g_n2, b_n2):
    raise NotImplementedError("write your pallas kernel here")


# speedup vs baseline: 1.2856x; 1.2856x over previous
import functools

import jax
import jax.numpy as jnp
from jax import lax
from jax.experimental import pallas as pl
from jax.experimental.pallas import tpu as pltpu

N = 10000
E = 320000
D = 128
DE = 16
H = 8
DH = D // H
DFF = 512

NB = 400
EB = 256


def _ln(x, g, b, eps=1e-5):
    mu = jnp.mean(x, axis=-1, keepdims=True)
    var = jnp.mean((x - mu) ** 2, axis=-1, keepdims=True)
    return g * (x - mu) * lax.rsqrt(var + eps) + b


def _dot(a, b):
    return jnp.dot(a, b, preferred_element_type=jnp.float32)


def _tables_body(x_ref, wd_ref, ws_ref, td_ref, ts_ref):
    x = x_ref[...]
    td_ref[...] = _dot(x, wd_ref[...])
    ts_ref[...] = _dot(x, ws_ref[...])


def _tables(x, w_dst, w_src):
    return pl.pallas_call(
        _tables_body,
        grid=(N // NB,),
        in_specs=[
            pl.BlockSpec((NB, D), lambda i: (i, 0)),
            pl.BlockSpec((D, 2 * D), lambda i: (0, 0)),
            pl.BlockSpec((D, D), lambda i: (0, 0)),
        ],
        out_specs=[
            pl.BlockSpec((NB, 2 * D), lambda i: (i, 0)),
            pl.BlockSpec((NB, D), lambda i: (i, 0)),
        ],
        out_shape=[
            jax.ShapeDtypeStruct((N, 2 * D), jnp.float32),
            jax.ShapeDtypeStruct((N, D), jnp.float32),
        ],
        compiler_params=pltpu.CompilerParams(
            dimension_semantics=("parallel",)),
    )(x, w_dst, w_src)


def _edge_body(g1_ref, g2_ref, ea_ref, wme_ref, bmem_ref, gmln_ref, bmln_ref,
               wk_ref, wv_ref, weu_ref, beu_ref, geln_ref, beln_ref,
               gen_ref, ben_ref, p_ref, pt_ref,
               w_ref, e_ref, uea_ref):
    g1 = g1_ref[...]
    a = g1[:, :D]
    q = g1[:, D:]
    ea = ea_ref[...]
    pre = a + g2_ref[...] + _dot(ea, wme_ref[...]) + bmem_ref[...]
    mem = jnp.maximum(_ln(pre, gmln_ref[...], bmln_ref[...]), 0.0)
    k = _dot(mem, wk_ref[...])
    v = _dot(mem, wv_ref[...])
    l16 = _dot(q * k, p_ref[...]) * (1.0 / (DH ** 0.5))
    e16 = jnp.exp(l16)
    w_ref[...] = _dot(e16, pt_ref[...]) * v
    e_ref[...] = e16
    de = jnp.maximum(
        _ln(_dot(mem, weu_ref[...]) + beu_ref[...], geln_ref[...], beln_ref[...]),
        0.0)
    uea_ref[...] = _ln(ea + de, gen_ref[...], ben_ref[...])


def _edge(g1, g2, edge_attr, w_me, b_mem, g_mln, b_mln, Wk, Wv,
          W_eu, b_eu, g_eln, b_eln, g_en, b_en, P, Pt):
    row = lambda i: (i, 0)
    full = lambda shape: pl.BlockSpec(shape, lambda i: (0, 0))
    return pl.pallas_call(
        _edge_body,
        grid=(E // EB,),
        in_specs=[
            pl.BlockSpec((EB, 2 * D), row),
            pl.BlockSpec((EB, D), row),
            pl.BlockSpec((EB, DE), row),
            full((DE, D)), full((1, D)), full((1, D)), full((1, D)),
            full((D, D)), full((D, D)),
            full((D, DE)), full((1, DE)), full((1, DE)), full((1, DE)),
            full((1, DE)), full((1, DE)),
            full((D, DE)), full((DE, D)),
        ],
        out_specs=[
            pl.BlockSpec((EB, D), row),
            pl.BlockSpec((EB, DE), row),
            pl.BlockSpec((EB, DE), row),
        ],
        out_shape=[
            jax.ShapeDtypeStruct((E, D), jnp.float32),
            jax.ShapeDtypeStruct((E, DE), jnp.float32),
            jax.ShapeDtypeStruct((E, DE), jnp.float32),
        ],
        compiler_params=pltpu.CompilerParams(
            dimension_semantics=("parallel",)),
    )(g1, g2, edge_attr, w_me, b_mem, g_mln, b_mln, Wk, Wv,
      W_eu, b_eu, g_eln, b_eln, g_en, b_en, P, Pt)


def _node_body(x_ref, sw_ref, se_ref, wo_ref, pt_ref, gn1_ref, bn1_ref,
               wf1_ref, bf1_ref, wf2_ref, bf2_ref, gn2_ref, bn2_ref, out_ref):
    sw = sw_ref[0] + sw_ref[1]
    se = se_ref[0] + se_ref[1]
    zb = _dot(se, pt_ref[...])
    aggr = _dot(sw / (zb + 1e-16), wo_ref[...])
    h = _ln(x_ref[...] + aggr, gn1_ref[...], bn1_ref[...])
    ff = _dot(jnp.maximum(_dot(h, wf1_ref[...]) + bf1_ref[...], 0.0),
              wf2_ref[...]) + bf2_ref[...]
    out_ref[...] = _ln(h + ff, gn2_ref[...], bn2_ref[...])


def _node(x, sw, se, Wo, Pt, g_n1, b_n1, W_f1, b_f1, W_f2, b_f2, g_n2, b_n2):
    full = lambda shape: pl.BlockSpec(shape, lambda i: (0, 0))
    return pl.pallas_call(
        _node_body,
        grid=(N // NB,),
        in_specs=[
            pl.BlockSpec((NB, D), lambda i: (i, 0)),
            pl.BlockSpec((2, NB, D), lambda i: (0, i, 0)),
            pl.BlockSpec((2, NB, DE), lambda i: (0, i, 0)),
            full((D, D)), full((DE, D)),
            full((1, D)), full((1, D)),
            full((D, DFF)), full((1, DFF)),
            full((DFF, D)), full((1, D)),
            full((1, D)), full((1, D)),
        ],
        out_specs=pl.BlockSpec((NB, D), lambda i: (i, 0)),
        out_shape=jax.ShapeDtypeStruct((N, D), jnp.float32),
        compiler_params=pltpu.CompilerParams(
            dimension_semantics=("parallel",)),
    )(x, sw, se, Wo, Pt, g_n1, b_n1, W_f1, b_f1, W_f2, b_f2, g_n2, b_n2)


def _gather(td, ts, idx_dst, idx_src):
    g1 = jnp.take(td, idx_dst, axis=0)
    g2 = jnp.take(ts, idx_src, axis=0)
    return g1, g2


def _scatter(w, e16, seg):
    sw = jax.ops.segment_sum(w, seg, num_segments=N)
    se = jax.ops.segment_sum(e16, seg, num_segments=N)
    sw = jnp.stack([sw, jnp.zeros_like(sw)])
    se = jnp.stack([se, jnp.zeros_like(se)])
    return sw, se


def kernel(x, edge_index, edge_attr, W_mem, b_mem, g_mln, b_mln, Wq, Wk, Wv, Wo,
           W_eu, b_eu, g_eln, b_eln, g_en, b_en, W_f1, b_f1, W_f2, b_f2,
           g_n1, b_n1, g_n2, b_n2):
    f32 = jnp.float32
    P = (lax.broadcasted_iota(jnp.int32, (D, DE), 0) // DH ==
         lax.broadcasted_iota(jnp.int32, (D, DE), 1)).astype(f32)
    Pt = P.T
    w_dst = jnp.concatenate([W_mem[:D], Wq], axis=1)
    w_src = W_mem[D:2 * D]
    w_me = W_mem[2 * D:]
    r = lambda a: a.reshape(1, -1)

    td, ts = _tables(x, w_dst, w_src)
    g1, g2 = _gather(td, ts, edge_index[1], edge_index[0])
    w, e16, uea = _edge(g1, g2, edge_attr, w_me, r(b_mem), r(g_mln), r(b_mln),
                        Wk, Wv, W_eu, r(b_eu), r(g_eln), r(b_eln),
                        r(g_en), r(b_en), P, Pt)
    sw, se = _scatter(w, e16, edge_index[1])
    out = _node(x, sw, se, Wo, Pt, r(g_n1), r(b_n1), W_f1, r(b_f1),
                W_f2, r(b_f2), r(g_n2), r(b_n2))
    return (out, uea)


# speedup vs baseline: 1.8259x; 1.4202x over previous
import functools

import jax
import jax.numpy as jnp
from jax import lax
from jax.experimental import pallas as pl
from jax.experimental.pallas import tpu as pltpu
from jax.experimental.pallas import tpu_sc as plsc

N = 10000
E = 320000
D = 128
DE = 16
H = 8
DH = D // H
DFF = 512

NB = 400
EB = 256


def _ln(x, g, b, eps=1e-5):
    mu = jnp.mean(x, axis=-1, keepdims=True)
    var = jnp.mean((x - mu) ** 2, axis=-1, keepdims=True)
    return g * (x - mu) * lax.rsqrt(var + eps) + b


def _dot(a, b):
    return jnp.dot(a, b, preferred_element_type=jnp.float32)


def _tables_body(x_ref, wd_ref, ws_ref, td_ref, ts_ref):
    x = x_ref[...]
    td_ref[...] = _dot(x, wd_ref[...])
    ts_ref[...] = _dot(x, ws_ref[...])


def _tables(x, w_dst, w_src):
    return pl.pallas_call(
        _tables_body,
        grid=(N // NB,),
        in_specs=[
            pl.BlockSpec((NB, D), lambda i: (i, 0)),
            pl.BlockSpec((D, 2 * D), lambda i: (0, 0)),
            pl.BlockSpec((D, D), lambda i: (0, 0)),
        ],
        out_specs=[
            pl.BlockSpec((NB, 2 * D), lambda i: (i, 0)),
            pl.BlockSpec((NB, D), lambda i: (i, 0)),
        ],
        out_shape=[
            jax.ShapeDtypeStruct((N, 2 * D), jnp.float32),
            jax.ShapeDtypeStruct((N, D), jnp.float32),
        ],
        compiler_params=pltpu.CompilerParams(
            dimension_semantics=("parallel",)),
    )(x, w_dst, w_src)


def _edge_body(g1_ref, g2_ref, ea_ref, wme_ref, bmem_ref, gmln_ref, bmln_ref,
               wk_ref, wv_ref, weu_ref, beu_ref, geln_ref, beln_ref,
               gen_ref, ben_ref, p_ref, pt_ref,
               w_ref, e_ref, uea_ref):
    g1 = g1_ref[...]
    a = g1[:, :D]
    q = g1[:, D:]
    ea = ea_ref[...]
    pre = a + g2_ref[...] + _dot(ea, wme_ref[...]) + bmem_ref[...]
    mem = jnp.maximum(_ln(pre, gmln_ref[...], bmln_ref[...]), 0.0)
    k = _dot(mem, wk_ref[...])
    v = _dot(mem, wv_ref[...])
    l16 = _dot(q * k, p_ref[...]) * (1.0 / (DH ** 0.5))
    e16 = jnp.exp(l16)
    w_ref[...] = _dot(e16, pt_ref[...]) * v
    e_ref[...] = e16
    de = jnp.maximum(
        _ln(_dot(mem, weu_ref[...]) + beu_ref[...], geln_ref[...], beln_ref[...]),
        0.0)
    uea_ref[...] = _ln(ea + de, gen_ref[...], ben_ref[...])


def _edge(g1, g2, edge_attr, w_me, b_mem, g_mln, b_mln, Wk, Wv,
          W_eu, b_eu, g_eln, b_eln, g_en, b_en, P, Pt):
    row = lambda i: (i, 0)
    full = lambda shape: pl.BlockSpec(shape, lambda i: (0, 0))
    return pl.pallas_call(
        _edge_body,
        grid=(E // EB,),
        in_specs=[
            pl.BlockSpec((EB, 2 * D), row),
            pl.BlockSpec((EB, D), row),
            pl.BlockSpec((EB, DE), row),
            full((DE, D)), full((1, D)), full((1, D)), full((1, D)),
            full((D, D)), full((D, D)),
            full((D, DE)), full((1, DE)), full((1, DE)), full((1, DE)),
            full((1, DE)), full((1, DE)),
            full((D, DE)), full((DE, D)),
        ],
        out_specs=[
            pl.BlockSpec((EB, D), row),
            pl.BlockSpec((EB, DE), row),
            pl.BlockSpec((EB, DE), row),
        ],
        out_shape=[
            jax.ShapeDtypeStruct((E, D), jnp.float32),
            jax.ShapeDtypeStruct((E, DE), jnp.float32),
            jax.ShapeDtypeStruct((E, DE), jnp.float32),
        ],
        compiler_params=pltpu.CompilerParams(
            dimension_semantics=("parallel",)),
    )(g1, g2, edge_attr, w_me, b_mem, g_mln, b_mln, Wk, Wv,
      W_eu, b_eu, g_eln, b_eln, g_en, b_en, P, Pt)


def _node_body(x_ref, sw_ref, se_ref, wo_ref, pt_ref, gn1_ref, bn1_ref,
               wf1_ref, bf1_ref, wf2_ref, bf2_ref, gn2_ref, bn2_ref, out_ref):
    sw = sw_ref[0] + sw_ref[1]
    se = se_ref[0] + se_ref[1]
    zb = _dot(se, pt_ref[...])
    aggr = _dot(sw / (zb + 1e-16), wo_ref[...])
    h = _ln(x_ref[...] + aggr, gn1_ref[...], bn1_ref[...])
    ff = _dot(jnp.maximum(_dot(h, wf1_ref[...]) + bf1_ref[...], 0.0),
              wf2_ref[...]) + bf2_ref[...]
    out_ref[...] = _ln(h + ff, gn2_ref[...], bn2_ref[...])


def _node(x, sw, se, Wo, Pt, g_n1, b_n1, W_f1, b_f1, W_f2, b_f2, g_n2, b_n2):
    full = lambda shape: pl.BlockSpec(shape, lambda i: (0, 0))
    return pl.pallas_call(
        _node_body,
        grid=(N // NB,),
        in_specs=[
            pl.BlockSpec((NB, D), lambda i: (i, 0)),
            pl.BlockSpec((2, NB, D), lambda i: (0, i, 0)),
            pl.BlockSpec((2, NB, DE), lambda i: (0, i, 0)),
            full((D, D)), full((DE, D)),
            full((1, D)), full((1, D)),
            full((D, DFF)), full((1, DFF)),
            full((DFF, D)), full((1, D)),
            full((1, D)), full((1, D)),
        ],
        out_specs=pl.BlockSpec((NB, D), lambda i: (i, 0)),
        out_shape=jax.ShapeDtypeStruct((N, D), jnp.float32),
        compiler_params=pltpu.CompilerParams(
            dimension_semantics=("parallel",)),
    )(x, sw, se, Wo, Pt, g_n1, b_n1, W_f1, b_f1, W_f2, b_f2, g_n2, b_n2)


GW = 80
SW = 80
NP = 10240


def _gather(td, ts, idx_dst, idx_src):
    mesh = plsc.VectorSubcoreMesh(core_axis_name="c", subcore_axis_name="s")
    ept = E // 32
    nstep = ept // GW

    @functools.partial(
        pl.kernel,
        out_type=[jax.ShapeDtypeStruct((E, 2 * D), jnp.float32),
                  jax.ShapeDtypeStruct((E, D), jnp.float32)],
        mesh=mesh,
        scratch_types=[pltpu.VMEM((GW, 2 * D), jnp.float32),
                       pltpu.VMEM((GW, D), jnp.float32),
                       pltpu.VMEM((GW,), jnp.int32),
                       pltpu.VMEM((GW,), jnp.int32)],
    )
    def k(td_hbm, ts_hbm, id_hbm, is_hbm, g1_hbm, g2_hbm,
          g1buf, g2buf, idbuf, isbuf):
        cid = lax.axis_index("c")
        sid = lax.axis_index("s")
        wid = sid * 2 + cid
        base0 = wid * ept

        @pl.loop(0, nstep)
        def _(j):
            b = base0 + j * GW
            sl = pl.ds(b, GW)
            pltpu.sync_copy(id_hbm.at[sl], idbuf)
            pltpu.sync_copy(is_hbm.at[sl], isbuf)
            pltpu.sync_copy(td_hbm.at[idbuf], g1buf)
            pltpu.sync_copy(ts_hbm.at[isbuf], g2buf)
            pltpu.sync_copy(g1buf, g1_hbm.at[sl])
            pltpu.sync_copy(g2buf, g2_hbm.at[sl])

    return k(td, ts, idx_dst.reshape(E), idx_src.reshape(E))


def _scatter_jnp(w, e16, seg):
    sw = jax.ops.segment_sum(w, seg, num_segments=NP)
    se = jax.ops.segment_sum(e16, seg, num_segments=NP)
    return (jnp.stack([sw, jnp.zeros_like(sw)]),
            jnp.stack([se, jnp.zeros_like(se)]))


def _scatter(w, e16, seg):
    mesh = plsc.VectorSubcoreMesh(core_axis_name="c", subcore_axis_name="s")
    rows = NP // 16
    ept = E // 32
    nstep = ept // SW
    zw = jnp.zeros((NP, D), jnp.float32)
    ze = jnp.zeros((NP, DE), jnp.float32)

    @functools.partial(
        pl.kernel,
        out_type=[jax.ShapeDtypeStruct((2, NP, D), jnp.float32),
                  jax.ShapeDtypeStruct((2, NP, DE), jnp.float32)],
        mesh=mesh,
        scratch_types=[pltpu.VMEM_SHARED((NP, D), jnp.float32),
                       pltpu.VMEM_SHARED((NP, DE), jnp.float32),
                       pltpu.VMEM((SW, D), jnp.float32),
                       pltpu.VMEM((SW, DE), jnp.float32),
                       pltpu.VMEM((SW,), jnp.int32)],
    )
    def k(w_hbm, e_hbm, i_hbm, zw_hbm, ze_hbm, ow_hbm, oe_hbm,
          accw, acce, wbuf, ebuf, ibuf):
        cid = lax.axis_index("c")
        sid = lax.axis_index("s")
        wid = sid * 2 + cid

        @pl.loop(0, rows, step=64)
        def _(j):
            zsl = pl.ds(sid * rows + j, 64)
            pltpu.sync_copy(zw_hbm.at[zsl], accw.at[zsl])
            pltpu.sync_copy(ze_hbm.at[zsl], acce.at[zsl])

        plsc.subcore_barrier()
        base0 = wid * ept

        @pl.loop(0, nstep)
        def _(j):
            b = base0 + j * SW
            pltpu.sync_copy(w_hbm.at[pl.ds(b, SW)], wbuf)
            pltpu.sync_copy(e_hbm.at[pl.ds(b, SW)], ebuf)
            pltpu.sync_copy(i_hbm.at[pl.ds(b, SW)], ibuf)
            pltpu.sync_copy(wbuf, accw.at[ibuf], add=True)
            pltpu.sync_copy(ebuf, acce.at[ibuf], add=True)

        plsc.subcore_barrier()

        @pl.loop(0, rows, step=64)
        def _(j):
            osl = pl.ds(sid * rows + j, 64)
            pltpu.sync_copy(accw.at[osl], ow_hbm.at[cid, osl])
            pltpu.sync_copy(acce.at[osl], oe_hbm.at[cid, osl])

    return k(w, e16, seg.reshape(E), zw, ze)


def kernel(x, edge_index, edge_attr, W_mem, b_mem, g_mln, b_mln, Wq, Wk, Wv, Wo,
           W_eu, b_eu, g_eln, b_eln, g_en, b_en, W_f1, b_f1, W_f2, b_f2,
           g_n1, b_n1, g_n2, b_n2):
    f32 = jnp.float32
    P = (lax.broadcasted_iota(jnp.int32, (D, DE), 0) // DH ==
         lax.broadcasted_iota(jnp.int32, (D, DE), 1)).astype(f32)
    Pt = P.T
    w_dst = jnp.concatenate([W_mem[:D], Wq], axis=1)
    w_src = W_mem[D:2 * D]
    w_me = W_mem[2 * D:]
    r = lambda a: a.reshape(1, -1)

    td, ts = _tables(x, w_dst, w_src)
    g1, g2 = _gather(td, ts, edge_index[1], edge_index[0])
    w, e16, uea = _edge(g1, g2, edge_attr, w_me, r(b_mem), r(g_mln), r(b_mln),
                        Wk, Wv, W_eu, r(b_eu), r(g_eln), r(b_eln),
                        r(g_en), r(b_en), P, Pt)
    sw, se = _scatter_jnp(w, e16, edge_index[1])
    out = _node(x, sw, se, Wo, Pt, r(g_n1), r(b_n1), W_f1, r(b_f1),
                W_f2, r(b_f2), r(g_n2), r(b_n2))
    return (out, uea)


# speedup vs baseline: 3.8871x; 2.1289x over previous
import functools

import jax
import jax.numpy as jnp
from jax import lax
from jax.experimental import pallas as pl
from jax.experimental.pallas import tpu as pltpu
from jax.experimental.pallas import tpu_sc as plsc

N = 10000
E = 320000
D = 128
DE = 16
H = 8
DH = D // H
DFF = 512

NB = 400
EB = 512


def _ln(x, g, b, eps=1e-5):
    mu = jnp.mean(x, axis=-1, keepdims=True)
    var = jnp.mean((x - mu) ** 2, axis=-1, keepdims=True)
    return g * (x - mu) * lax.rsqrt(var + eps) + b


def _dot(a, b):
    return jnp.dot(a, b, preferred_element_type=jnp.float32)


def _ln_mx(x, g, b, eps=1e-5):
    n = x.shape[-1]
    j = jnp.full((n, n), 1.0 / n, dtype=jnp.float32)
    mu = _dot(x, j)
    xc = x - mu
    var = _dot(xc * xc, j)
    return g * xc * lax.rsqrt(var + eps) + b


def _tables_body(x_ref, wd_ref, ws_ref, td_ref, ts_ref):
    x = x_ref[...]
    td_ref[...] = _dot(x, wd_ref[...])
    ts_ref[...] = _dot(x, ws_ref[...])


def _tables(x, w_dst, w_src):
    return pl.pallas_call(
        _tables_body,
        grid=(N // NB,),
        in_specs=[
            pl.BlockSpec((NB, D), lambda i: (i, 0)),
            pl.BlockSpec((D, 2 * D), lambda i: (0, 0)),
            pl.BlockSpec((D, D), lambda i: (0, 0)),
        ],
        out_specs=[
            pl.BlockSpec((NB, 2 * D), lambda i: (i, 0)),
            pl.BlockSpec((NB, D), lambda i: (i, 0)),
        ],
        out_shape=[
            jax.ShapeDtypeStruct((N, 2 * D), jnp.float32),
            jax.ShapeDtypeStruct((N, D), jnp.float32),
        ],
        compiler_params=pltpu.CompilerParams(
            dimension_semantics=("parallel",)),
    )(x, w_dst, w_src)


def _edge_body(g1_ref, g2_ref, ea_ref, wme_ref, bmem_ref, gmln_ref, bmln_ref,
               wk_ref, wv_ref, weu_ref, beu_ref, geln_ref, beln_ref,
               gen_ref, ben_ref, p_ref, pt_ref,
               w_ref, e_ref, uea_ref):
    g1 = g1_ref[...]
    a = g1[:, :D]
    q = g1[:, D:]
    ea = ea_ref[...]
    pre = a + g2_ref[...] + _dot(ea, wme_ref[...]) + bmem_ref[...]
    mem = jnp.maximum(_ln_mx(pre, gmln_ref[...], bmln_ref[...]), 0.0)
    k = _dot(mem, wk_ref[...])
    v = _dot(mem, wv_ref[...])
    l16 = _dot(q * k, p_ref[...]) * (1.0 / (DH ** 0.5))
    eb = jnp.exp(_dot(l16, pt_ref[...]))
    w_ref[...] = eb * v
    e_ref[...] = eb
    de = jnp.maximum(
        _ln_mx(_dot(mem, weu_ref[...]) + beu_ref[...], geln_ref[...],
               beln_ref[...]), 0.0)
    uea_ref[...] = _ln_mx(ea + de, gen_ref[...], ben_ref[...])


def _edge(g1, g2, edge_attr, w_me, b_mem, g_mln, b_mln, Wk, Wv,
          W_eu, b_eu, g_eln, b_eln, g_en, b_en, P, Pt):
    row = lambda i: (i, 0)
    full = lambda shape: pl.BlockSpec(shape, lambda i: (0, 0))
    return pl.pallas_call(
        _edge_body,
        grid=(E // EB,),
        in_specs=[
            pl.BlockSpec((EB, 2 * D), row),
            pl.BlockSpec((EB, D), row),
            pl.BlockSpec((EB, DE), row),
            full((DE, D)), full((1, D)), full((1, D)), full((1, D)),
            full((D, D)), full((D, D)),
            full((D, DE)), full((1, DE)), full((1, DE)), full((1, DE)),
            full((1, DE)), full((1, DE)),
            full((D, DE)), full((DE, D)),
        ],
        out_specs=[
            pl.BlockSpec((EB, D), row),
            pl.BlockSpec((EB, D), row),
            pl.BlockSpec((EB, DE), row),
        ],
        out_shape=[
            jax.ShapeDtypeStruct((E, D), jnp.float32),
            jax.ShapeDtypeStruct((E, D), jnp.float32),
            jax.ShapeDtypeStruct((E, DE), jnp.float32),
        ],
        compiler_params=pltpu.CompilerParams(
            dimension_semantics=("parallel",)),
    )(g1, g2, edge_attr, w_me, b_mem, g_mln, b_mln, Wk, Wv,
      W_eu, b_eu, g_eln, b_eln, g_en, b_en, P, Pt)


def _node_body(x_ref, sw_ref, se_ref, wo_ref, gn1_ref, bn1_ref,
               wf1_ref, bf1_ref, wf2_ref, bf2_ref, gn2_ref, bn2_ref, out_ref):
    sw = sw_ref[0] + sw_ref[1]
    zb = se_ref[0] + se_ref[1]
    aggr = _dot(sw / (zb + 1e-16), wo_ref[...])
    h = _ln(x_ref[...] + aggr, gn1_ref[...], bn1_ref[...])
    ff = _dot(jnp.maximum(_dot(h, wf1_ref[...]) + bf1_ref[...], 0.0),
              wf2_ref[...]) + bf2_ref[...]
    out_ref[...] = _ln(h + ff, gn2_ref[...], bn2_ref[...])


def _node(x, sw, se, Wo, g_n1, b_n1, W_f1, b_f1, W_f2, b_f2, g_n2, b_n2):
    full = lambda shape: pl.BlockSpec(shape, lambda i: (0, 0))
    return pl.pallas_call(
        _node_body,
        grid=(N // NB,),
        in_specs=[
            pl.BlockSpec((NB, D), lambda i: (i, 0)),
            pl.BlockSpec((2, NB, D), lambda i: (0, i, 0)),
            pl.BlockSpec((2, NB, D), lambda i: (0, i, 0)),
            full((D, D)),
            full((1, D)), full((1, D)),
            full((D, DFF)), full((1, DFF)),
            full((DFF, D)), full((1, D)),
            full((1, D)), full((1, D)),
        ],
        out_specs=pl.BlockSpec((NB, D), lambda i: (i, 0)),
        out_shape=jax.ShapeDtypeStruct((N, D), jnp.float32),
        compiler_params=pltpu.CompilerParams(
            dimension_semantics=("parallel",)),
    )(x, sw, se, Wo, g_n1, b_n1, W_f1, b_f1, W_f2, b_f2, g_n2, b_n2)


GW = 80
SW = 80
NP = 10240


def _gather(td, ts, idx_dst, idx_src):
    mesh = plsc.VectorSubcoreMesh(core_axis_name="c", subcore_axis_name="s")
    ept = E // 32
    nstep = ept // GW

    @functools.partial(
        pl.kernel,
        out_type=[jax.ShapeDtypeStruct((E, 2 * D), jnp.float32),
                  jax.ShapeDtypeStruct((E, D), jnp.float32)],
        mesh=mesh,
        scratch_types=[pltpu.VMEM((GW, 2 * D), jnp.float32),
                       pltpu.VMEM((GW, 2 * D), jnp.float32),
                       pltpu.VMEM((GW, D), jnp.float32),
                       pltpu.VMEM((GW, D), jnp.float32),
                       pltpu.VMEM((ept,), jnp.int32),
                       pltpu.VMEM((ept,), jnp.int32),
                       pltpu.SemaphoreType.DMA,
                       pltpu.SemaphoreType.DMA],
    )
    def k(td_hbm, ts_hbm, id_hbm, is_hbm, g1_hbm, g2_hbm,
          g1b0, g1b1, g2b0, g2b1, idall, isall, semA, semB):
        cid = lax.axis_index("c")
        sid = lax.axis_index("s")
        wid = sid * 2 + cid
        base0 = wid * ept
        pltpu.sync_copy(id_hbm.at[pl.ds(base0, ept)], idall)
        pltpu.sync_copy(is_hbm.at[pl.ds(base0, ept)], isall)

        def start(c, g1b, g2b, sem):
            sl = pl.ds(c * GW, GW)
            pltpu.async_copy(td_hbm.at[idall.at[sl]], g1b, sem)
            pltpu.async_copy(ts_hbm.at[isall.at[sl]], g2b, sem)

        def finish(c, g1b, g2b, sem):
            sl = pl.ds(c * GW, GW)
            hsl = pl.ds(base0 + c * GW, GW)
            pltpu.make_async_copy(td_hbm.at[idall.at[sl]], g1b, sem).wait()
            pltpu.make_async_copy(ts_hbm.at[isall.at[sl]], g2b, sem).wait()
            pltpu.sync_copy(g1b, g1_hbm.at[hsl])
            pltpu.sync_copy(g2b, g2_hbm.at[hsl])

        start(0, g1b0, g2b0, semA)

        @pl.loop(0, (nstep - 1) // 2)
        def _(j2):
            c0 = 2 * j2
            start(c0 + 1, g1b1, g2b1, semB)
            finish(c0, g1b0, g2b0, semA)
            start(c0 + 2, g1b0, g2b0, semA)
            finish(c0 + 1, g1b1, g2b1, semB)

        finish(nstep - 1, g1b0, g2b0, semA)

    return k(td, ts, idx_dst.reshape(E), idx_src.reshape(E))


def _scatter128(vals, seg):
    mesh = plsc.VectorSubcoreMesh(core_axis_name="c", subcore_axis_name="s")
    rows = NP // 16
    ept = E // 32
    nstep = ept // SW
    zw = jnp.zeros((64, D), jnp.float32)
    ar = jnp.arange(NP, dtype=jnp.int32)

    @functools.partial(
        pl.kernel,
        out_type=jax.ShapeDtypeStruct((2, NP, D), jnp.float32),
        mesh=mesh,
        scratch_types=[pltpu.VMEM_SHARED((NP, D), jnp.float32),
                       pltpu.VMEM((SW, D), jnp.float32),
                       pltpu.VMEM((SW, D), jnp.float32),
                       pltpu.VMEM((SW,), jnp.int32),
                       pltpu.VMEM((SW,), jnp.int32),
                       pltpu.VMEM((64, D), jnp.float32),
                       pltpu.VMEM((64,), jnp.int32),
                       pltpu.SemaphoreType.DMA,
                       pltpu.SemaphoreType.DMA],
    )
    def k(w_hbm, i_hbm, zw_hbm, ar_hbm, ow_hbm, accw,
          wb0, wb1, ib0, ib1, tw, rbuf, semA, semB):
        cid = lax.axis_index("c")
        sid = lax.axis_index("s")
        wid = sid * 2 + cid
        pltpu.sync_copy(zw_hbm, tw)

        @pl.loop(0, rows, step=64)
        def _(j):
            pltpu.sync_copy(ar_hbm.at[pl.ds(sid * rows + j, 64)], rbuf)
            pltpu.sync_copy(tw, accw.at[rbuf])

        plsc.subcore_barrier()
        base0 = wid * ept

        def start(c, wb, ib, sem):
            hsl = pl.ds(base0 + c * SW, SW)
            pltpu.async_copy(w_hbm.at[hsl], wb, sem)
            pltpu.async_copy(i_hbm.at[hsl], ib, sem)

        def finish(c, wb, ib, sem):
            hsl = pl.ds(base0 + c * SW, SW)
            pltpu.make_async_copy(w_hbm.at[hsl], wb, sem).wait()
            pltpu.make_async_copy(i_hbm.at[hsl], ib, sem).wait()
            pltpu.sync_copy(wb, accw.at[ib], add=True)

        start(0, wb0, ib0, semA)

        @pl.loop(0, (nstep - 1) // 2)
        def _(j2):
            c0 = 2 * j2
            start(c0 + 1, wb1, ib1, semB)
            finish(c0, wb0, ib0, semA)
            start(c0 + 2, wb0, ib0, semA)
            finish(c0 + 1, wb1, ib1, semB)

        finish(nstep - 1, wb0, ib0, semA)
        plsc.subcore_barrier()

        @pl.loop(0, rows, step=64)
        def _(j):
            osl = pl.ds(sid * rows + j, 64)
            pltpu.sync_copy(ar_hbm.at[pl.ds(sid * rows + j, 64)], rbuf)
            pltpu.sync_copy(accw.at[rbuf], tw)
            pltpu.sync_copy(tw, ow_hbm.at[cid, osl])

    return k(vals, seg.reshape(E), zw, ar)


def kernel(x, edge_index, edge_attr, W_mem, b_mem, g_mln, b_mln, Wq, Wk, Wv, Wo,
           W_eu, b_eu, g_eln, b_eln, g_en, b_en, W_f1, b_f1, W_f2, b_f2,
           g_n1, b_n1, g_n2, b_n2):
    f32 = jnp.float32
    P = (lax.broadcasted_iota(jnp.int32, (D, DE), 0) // DH ==
         lax.broadcasted_iota(jnp.int32, (D, DE), 1)).astype(f32)
    Pt = P.T
    w_dst = jnp.concatenate([W_mem[:D], Wq], axis=1)
    w_src = W_mem[D:2 * D]
    w_me = W_mem[2 * D:]
    r = lambda a: a.reshape(1, -1)

    td, ts = _tables(x, w_dst, w_src)
    g1, g2 = _gather(td, ts, edge_index[1], edge_index[0])
    w, eb, uea = _edge(g1, g2, edge_attr, w_me, r(b_mem), r(g_mln), r(b_mln),
                       Wk, Wv, W_eu, r(b_eu), r(g_eln), r(b_eln),
                       r(g_en), r(b_en), P, Pt)
    sw = _scatter128(w, edge_index[1])
    se = _scatter128(eb, edge_index[1])
    out = _node(x, sw, se, Wo, r(g_n1), r(b_n1), W_f1, r(b_f1),
                W_f2, r(b_f2), r(g_n2), r(b_n2))
    return (out, uea)


# speedup vs baseline: 4.0087x; 1.0313x over previous
import functools

import jax
import jax.numpy as jnp
from jax import lax
from jax.experimental import pallas as pl
from jax.experimental.pallas import tpu as pltpu
from jax.experimental.pallas import tpu_sc as plsc

N = 10000
E = 320000
D = 128
DE = 16
H = 8
DH = D // H
DFF = 512

NB = 400
EB = 512


def _ln(x, g, b, eps=1e-5):
    mu = jnp.mean(x, axis=-1, keepdims=True)
    var = jnp.mean((x - mu) ** 2, axis=-1, keepdims=True)
    return g * (x - mu) * lax.rsqrt(var + eps) + b


def _dot(a, b):
    return jnp.dot(a, b, preferred_element_type=jnp.float32)


def _tables_body(x_ref, wd_ref, ws_ref, td_ref, ts_ref):
    x = x_ref[...]
    td_ref[...] = _dot(x, wd_ref[...])
    ts_ref[...] = _dot(x, ws_ref[...])


def _tables(x, w_dst, w_src):
    return pl.pallas_call(
        _tables_body,
        grid=(N // NB,),
        in_specs=[
            pl.BlockSpec((NB, D), lambda i: (i, 0)),
            pl.BlockSpec((D, 2 * D), lambda i: (0, 0)),
            pl.BlockSpec((D, D), lambda i: (0, 0)),
        ],
        out_specs=[
            pl.BlockSpec((NB, 2 * D), lambda i: (i, 0)),
            pl.BlockSpec((NB, D), lambda i: (i, 0)),
        ],
        out_shape=[
            jax.ShapeDtypeStruct((N, 2 * D), jnp.float32),
            jax.ShapeDtypeStruct((N, D), jnp.float32),
        ],
        compiler_params=pltpu.CompilerParams(
            dimension_semantics=("parallel",)),
    )(x, w_dst, w_src)


def _edge_body(g1_ref, g2_ref, ea_ref, wme_ref, bmem_ref, gmln_ref, bmln_ref,
               wk_ref, wv_ref, weu_ref, beu_ref, geln_ref, beln_ref,
               gen_ref, ben_ref, p_ref, pt_ref,
               w_ref, e_ref, uea_ref):
    g1 = g1_ref[...]
    a = g1[:, :D]
    q = g1[:, D:]
    ea = ea_ref[...]
    pre = a + g2_ref[...] + _dot(ea, wme_ref[...]) + bmem_ref[...]
    mem = jnp.maximum(_ln(pre, gmln_ref[...], bmln_ref[...]), 0.0)
    k = _dot(mem, wk_ref[...])
    v = _dot(mem, wv_ref[...])
    l16 = _dot(q * k, p_ref[...]) * (1.0 / (DH ** 0.5))
    eb = jnp.exp(_dot(l16, pt_ref[...]))
    w_ref[...] = eb * v
    e_ref[...] = eb
    de = jnp.maximum(
        _ln(_dot(mem, weu_ref[...]) + beu_ref[...], geln_ref[...], beln_ref[...]),
        0.0)
    uea_ref[...] = _ln(ea + de, gen_ref[...], ben_ref[...])


def _edge(g1, g2, edge_attr, w_me, b_mem, g_mln, b_mln, Wk, Wv,
          W_eu, b_eu, g_eln, b_eln, g_en, b_en, P, Pt):
    row = lambda i: (i, 0)
    full = lambda shape: pl.BlockSpec(shape, lambda i: (0, 0))
    return pl.pallas_call(
        _edge_body,
        grid=(E // EB,),
        in_specs=[
            pl.BlockSpec((EB, 2 * D), row),
            pl.BlockSpec((EB, D), row),
            pl.BlockSpec((EB, DE), row),
            full((DE, D)), full((1, D)), full((1, D)), full((1, D)),
            full((D, D)), full((D, D)),
            full((D, DE)), full((1, DE)), full((1, DE)), full((1, DE)),
            full((1, DE)), full((1, DE)),
            full((D, DE)), full((DE, D)),
        ],
        out_specs=[
            pl.BlockSpec((EB, D), row),
            pl.BlockSpec((EB, D), row),
            pl.BlockSpec((EB, DE), row),
        ],
        out_shape=[
            jax.ShapeDtypeStruct((E, D), jnp.float32),
            jax.ShapeDtypeStruct((E, D), jnp.float32),
            jax.ShapeDtypeStruct((E, DE), jnp.float32),
        ],
        compiler_params=pltpu.CompilerParams(
            dimension_semantics=("parallel",)),
    )(g1, g2, edge_attr, w_me, b_mem, g_mln, b_mln, Wk, Wv,
      W_eu, b_eu, g_eln, b_eln, g_en, b_en, P, Pt)


def _node_body(x_ref, sw_ref, se_ref, wo_ref, gn1_ref, bn1_ref,
               wf1_ref, bf1_ref, wf2_ref, bf2_ref, gn2_ref, bn2_ref, out_ref):
    sw = sw_ref[0] + sw_ref[1]
    zb = se_ref[0] + se_ref[1]
    aggr = _dot(sw / (zb + 1e-16), wo_ref[...])
    h = _ln(x_ref[...] + aggr, gn1_ref[...], bn1_ref[...])
    ff = _dot(jnp.maximum(_dot(h, wf1_ref[...]) + bf1_ref[...], 0.0),
              wf2_ref[...]) + bf2_ref[...]
    out_ref[...] = _ln(h + ff, gn2_ref[...], bn2_ref[...])


def _node(x, sw, se, Wo, g_n1, b_n1, W_f1, b_f1, W_f2, b_f2, g_n2, b_n2):
    full = lambda shape: pl.BlockSpec(shape, lambda i: (0, 0))
    return pl.pallas_call(
        _node_body,
        grid=(N // NB,),
        in_specs=[
            pl.BlockSpec((NB, D), lambda i: (i, 0)),
            pl.BlockSpec((2, NB, D), lambda i: (0, i, 0)),
            pl.BlockSpec((2, NB, D), lambda i: (0, i, 0)),
            full((D, D)),
            full((1, D)), full((1, D)),
            full((D, DFF)), full((1, DFF)),
            full((DFF, D)), full((1, D)),
            full((1, D)), full((1, D)),
        ],
        out_specs=pl.BlockSpec((NB, D), lambda i: (i, 0)),
        out_shape=jax.ShapeDtypeStruct((N, D), jnp.float32),
        compiler_params=pltpu.CompilerParams(
            dimension_semantics=("parallel",)),
    )(x, sw, se, Wo, g_n1, b_n1, W_f1, b_f1, W_f2, b_f2, g_n2, b_n2)


GW = 80
SW = 80
NP = 10240


def _gather(td, ts, idx_dst, idx_src):
    mesh = plsc.VectorSubcoreMesh(core_axis_name="c", subcore_axis_name="s")
    ept = E // 32
    nstep = ept // GW

    @functools.partial(
        pl.kernel,
        out_type=[jax.ShapeDtypeStruct((E, 2 * D), jnp.float32),
                  jax.ShapeDtypeStruct((E, D), jnp.float32)],
        mesh=mesh,
        scratch_types=[pltpu.VMEM((GW, 2 * D), jnp.float32),
                       pltpu.VMEM((GW, 2 * D), jnp.float32),
                       pltpu.VMEM((GW, D), jnp.float32),
                       pltpu.VMEM((GW, D), jnp.float32),
                       pltpu.VMEM((ept,), jnp.int32),
                       pltpu.VMEM((ept,), jnp.int32),
                       pltpu.SemaphoreType.DMA,
                       pltpu.SemaphoreType.DMA],
    )
    def k(td_hbm, ts_hbm, id_hbm, is_hbm, g1_hbm, g2_hbm,
          g1b0, g1b1, g2b0, g2b1, idall, isall, semA, semB):
        cid = lax.axis_index("c")
        sid = lax.axis_index("s")
        wid = sid * 2 + cid
        base0 = wid * ept
        pltpu.sync_copy(id_hbm.at[pl.ds(base0, ept)], idall)
        pltpu.sync_copy(is_hbm.at[pl.ds(base0, ept)], isall)

        def start(c, g1b, g2b, sem):
            sl = pl.ds(c * GW, GW)
            pltpu.async_copy(td_hbm.at[idall.at[sl]], g1b, sem)
            pltpu.async_copy(ts_hbm.at[isall.at[sl]], g2b, sem)

        def finish(c, g1b, g2b, sem):
            sl = pl.ds(c * GW, GW)
            hsl = pl.ds(base0 + c * GW, GW)
            pltpu.make_async_copy(td_hbm.at[idall.at[sl]], g1b, sem).wait()
            pltpu.make_async_copy(ts_hbm.at[isall.at[sl]], g2b, sem).wait()
            pltpu.sync_copy(g1b, g1_hbm.at[hsl])
            pltpu.sync_copy(g2b, g2_hbm.at[hsl])

        start(0, g1b0, g2b0, semA)

        @pl.loop(0, (nstep - 1) // 2)
        def _(j2):
            c0 = 2 * j2
            start(c0 + 1, g1b1, g2b1, semB)
            finish(c0, g1b0, g2b0, semA)
            start(c0 + 2, g1b0, g2b0, semA)
            finish(c0 + 1, g1b1, g2b1, semB)

        finish(nstep - 1, g1b0, g2b0, semA)

    return k(td, ts, idx_dst.reshape(E), idx_src.reshape(E))


def _scatter128(vals, seg):
    mesh = plsc.VectorSubcoreMesh(core_axis_name="c", subcore_axis_name="s")
    rows = NP // 16
    ept = E // 32
    nstep = ept // SW
    zw = jnp.zeros((64, D), jnp.float32)
    ar = jnp.arange(NP, dtype=jnp.int32)

    @functools.partial(
        pl.kernel,
        out_type=jax.ShapeDtypeStruct((2, NP, D), jnp.float32),
        mesh=mesh,
        scratch_types=[pltpu.VMEM_SHARED((NP, D), jnp.float32),
                       pltpu.VMEM((SW, D), jnp.float32),
                       pltpu.VMEM((SW, D), jnp.float32),
                       pltpu.VMEM((SW,), jnp.int32),
                       pltpu.VMEM((SW,), jnp.int32),
                       pltpu.VMEM((64, D), jnp.float32),
                       pltpu.VMEM((64,), jnp.int32),
                       pltpu.SemaphoreType.DMA,
                       pltpu.SemaphoreType.DMA],
    )
    def k(w_hbm, i_hbm, zw_hbm, ar_hbm, ow_hbm, accw,
          wb0, wb1, ib0, ib1, tw, rbuf, semA, semB):
        cid = lax.axis_index("c")
        sid = lax.axis_index("s")
        wid = sid * 2 + cid
        pltpu.sync_copy(zw_hbm, tw)

        @pl.loop(0, rows, step=64)
        def _(j):
            pltpu.sync_copy(ar_hbm.at[pl.ds(sid * rows + j, 64)], rbuf)
            pltpu.sync_copy(tw, accw.at[rbuf])

        plsc.subcore_barrier()
        base0 = wid * ept

        def start(c, wb, ib, sem):
            hsl = pl.ds(base0 + c * SW, SW)
            pltpu.async_copy(w_hbm.at[hsl], wb, sem)
            pltpu.async_copy(i_hbm.at[hsl], ib, sem)

        def finish(c, wb, ib, sem):
            hsl = pl.ds(base0 + c * SW, SW)
            pltpu.make_async_copy(w_hbm.at[hsl], wb, sem).wait()
            pltpu.make_async_copy(i_hbm.at[hsl], ib, sem).wait()
            pltpu.sync_copy(wb, accw.at[ib], add=True)

        start(0, wb0, ib0, semA)

        @pl.loop(0, (nstep - 1) // 2)
        def _(j2):
            c0 = 2 * j2
            start(c0 + 1, wb1, ib1, semB)
            finish(c0, wb0, ib0, semA)
            start(c0 + 2, wb0, ib0, semA)
            finish(c0 + 1, wb1, ib1, semB)

        finish(nstep - 1, wb0, ib0, semA)
        plsc.subcore_barrier()

        @pl.loop(0, rows, step=64)
        def _(j):
            osl = pl.ds(sid * rows + j, 64)
            pltpu.sync_copy(ar_hbm.at[pl.ds(sid * rows + j, 64)], rbuf)
            pltpu.sync_copy(accw.at[rbuf], tw)
            pltpu.sync_copy(tw, ow_hbm.at[cid, osl])

    return k(vals, seg.reshape(E), zw, ar)


def kernel(x, edge_index, edge_attr, W_mem, b_mem, g_mln, b_mln, Wq, Wk, Wv, Wo,
           W_eu, b_eu, g_eln, b_eln, g_en, b_en, W_f1, b_f1, W_f2, b_f2,
           g_n1, b_n1, g_n2, b_n2):
    f32 = jnp.float32
    P = (lax.broadcasted_iota(jnp.int32, (D, DE), 0) // DH ==
         lax.broadcasted_iota(jnp.int32, (D, DE), 1)).astype(f32)
    Pt = P.T
    w_dst = jnp.concatenate([W_mem[:D], Wq], axis=1)
    w_src = W_mem[D:2 * D]
    w_me = W_mem[2 * D:]
    r = lambda a: a.reshape(1, -1)

    td, ts = _tables(x, w_dst, w_src)
    g1, g2 = _gather(td, ts, edge_index[1], edge_index[0])
    w, eb, uea = _edge(g1, g2, edge_attr, w_me, r(b_mem), r(g_mln), r(b_mln),
                       Wk, Wv, W_eu, r(b_eu), r(g_eln), r(b_eln),
                       r(g_en), r(b_en), P, Pt)
    sw = _scatter128(w, edge_index[1])
    se = _scatter128(eb, edge_index[1])
    out = _node(x, sw, se, Wo, r(g_n1), r(b_n1), W_f1, r(b_f1),
                W_f2, r(b_f2), r(g_n2), r(b_n2))
    return (out, uea)
